# baseline (speedup 1.0000x reference)
import functools

import jax
import jax.numpy as jnp
from jax import lax
from jax.experimental import pallas as pl
from jax.experimental.pallas import tpu as pltpu
from jax.experimental.pallas import tpu_sc as plsc

KCAP = 4
DD = 32
D = 128
NB = 128
C = 512
BIGC = 16
BIG = BIGC * C
ROUTE_ITERS = 6
SC_G = 512
SC_WORKERS = 32


def _sel(i, j):
    return (i == j).astype(jnp.float32)


def _mk_consts():
    w1 = _sel(lax.broadcasted_iota(jnp.int32, (D, D), 0) // KCAP,
              lax.broadcasted_iota(jnp.int32, (D, D), 1) % DD)
    g = _sel(lax.broadcasted_iota(jnp.int32, (D, KCAP), 0) // DD,
             lax.broadcasted_iota(jnp.int32, (D, KCAP), 1))
    gt = _sel(lax.broadcasted_iota(jnp.int32, (KCAP, D), 0),
              lax.broadcasted_iota(jnp.int32, (KCAP, D), 1) // DD)
    return w1, g, gt


def _group_normalize(v, g, gt):
    nrm2 = jnp.dot(v * v, g, preferred_element_type=jnp.float32)
    inv = 1.0 / jnp.maximum(jnp.sqrt(nrm2), 1e-12)
    return v * jnp.dot(inv, gt, preferred_element_type=jnp.float32)


def _norm_body(x_ref, o_ref):
    _, g, gt = _mk_consts()
    o_ref[...] = _group_normalize(x_ref[...], g, gt)


def _normalize(x_pad):
    return pl.pallas_call(
        _norm_body,
        out_shape=jax.ShapeDtypeStruct(x_pad.shape, jnp.float32),
    )(x_pad)


def _gather_rows(table, idx):
    mpad = idx.shape[0]
    per_w = mpad // SC_WORKERS
    nchunk = per_w // SC_G
    mesh = plsc.VectorSubcoreMesh(core_axis_name="c", subcore_axis_name="s")

    @functools.partial(
        pl.kernel,
        mesh=mesh,
        out_type=jax.ShapeDtypeStruct((mpad, D), jnp.float32),
        scratch_types=[
            pltpu.VMEM((SC_G,), jnp.int32),
            pltpu.VMEM((SC_G, D), jnp.float32),
            pltpu.SemaphoreType.DMA,
        ],
    )
    def sc_gather(table_hbm, idx_hbm, out_hbm, idx_v, rows_v, sem):
        wid = lax.axis_index("s") * 2 + lax.axis_index("c")
        base = wid * per_w

        @pl.loop(0, nchunk)
        def _(i):
            off = base + i * SC_G
            pltpu.sync_copy(idx_hbm.at[pl.ds(off, SC_G)], idx_v)
            pltpu.async_copy(table_hbm.at[idx_v], rows_v, sem).wait()
            pltpu.sync_copy(rows_v, out_hbm.at[pl.ds(off, SC_G)])

    return sc_gather(table, idx)


def _routing_body(starts_ref, xn_ref, z_hbm, trg_hbm, out_ref,
                  z_buf, trg_buf, dma_sem):
    b = pl.program_id(0)
    start = pl.multiple_of((starts_ref[b] // NB) * NB, NB)
    n_e = starts_ref[b + 1] - start
    nch = lax.div(n_e + (C - 1), C)
    npair = lax.div(nch + 1, 2)
    fits = n_e <= BIG

    w1, g, gt = _mk_consts()
    base_n = b * NB
    iota_n = lax.broadcasted_iota(jnp.int32, (NB, C), 0) + base_n

    x_blk = xn_ref[...]
    c = x_blk

    for t in range(ROUTE_ITERS):
        cst = jnp.dot(c, w1, preferred_element_type=jnp.float32)
        cst16 = cst.astype(jnp.bfloat16)

        def start_dma(i, slot):
            cp_z = pltpu.make_async_copy(
                z_hbm.at[pl.ds(start + i * C, C), :],
                z_buf.at[slot], dma_sem)
            cp_t = pltpu.make_async_copy(
                trg_hbm.at[:, pl.ds(start + i * C, C)],
                trg_buf.at[slot], dma_sem)
            cp_z.start()
            cp_t.start()
            return cp_z, cp_t

        def chunk_math(slot, acc):
            z_c = z_buf[slot]
            trg_c = trg_buf[slot]
            ot = _sel(iota_n, trg_c)
            ot16 = ot.astype(jnp.bfloat16)
            csg = lax.dot_general(ot16, cst16, (((0,), (0,)), ((), ())),
                                  preferred_element_type=jnp.float32)
            prod = z_c * csg
            p4 = jnp.dot(prod, g, preferred_element_type=jnp.float32)
            e4 = jnp.exp(p4)
            p4n = e4 / jnp.sum(e4, axis=1, keepdims=True)
            pt = jnp.dot(p4n, gt, preferred_element_type=jnp.float32)
            w = (pt * z_c).astype(jnp.bfloat16)
            return acc + jnp.dot(ot16, w, preferred_element_type=jnp.float32)

        def pair_body(j, acc, do_dma):
            i0 = 2 * j
            s0 = jnp.where(fits, i0, 0)
            s1 = jnp.where(fits, i0 + 1, 1)
            if do_dma:
                cps = start_dma(i0, s0) + start_dma(i0 + 1, s1)
                for cp in cps:
                    cp.wait()
            acc = chunk_math(s0, acc)
            return chunk_math(s1, acc)

        if t == 0:
            body = lambda j, acc: pair_body(j, acc, True)
        else:
            def body(j, acc):
                return lax.cond(fits,
                                lambda: pair_body(j, acc, False),
                                lambda: pair_body(j, acc, True))
        acc = lax.fori_loop(0, npair, body, jnp.zeros((NB, D), jnp.float32))
        c = x_blk + acc
        if t < ROUTE_ITERS - 1:
            c = _group_normalize(c, g, gt)

    out_ref[...] = c


def _routing(starts, xn_pad, z, trg_pad2d, npad):
    grid_spec = pltpu.PrefetchScalarGridSpec(
        num_scalar_prefetch=1,
        grid=(npad // NB,),
        in_specs=[
            pl.BlockSpec((NB, D), lambda b, s: (b, 0)),
            pl.BlockSpec(memory_space=pl.ANY),
            pl.BlockSpec(memory_space=pl.ANY),
        ],
        out_specs=pl.BlockSpec((NB, D), lambda b, s: (b, 0)),
        scratch_shapes=[
            pltpu.VMEM((BIGC, C, D), jnp.float32),
            pltpu.VMEM((BIGC, 1, C), jnp.int32),
            pltpu.SemaphoreType.DMA,
        ],
    )
    return pl.pallas_call(
        _routing_body,
        grid_spec=grid_spec,
        out_shape=jax.ShapeDtypeStruct((npad, D), jnp.float32),
    )(starts, xn_pad, z, trg_pad2d)


def kernel(x, edge_index):
    n, d = x.shape
    assert d == D
    src = edge_index[0]
    trg = edge_index[1]
    m = src.shape[0]

    nblk = -(-n // NB)
    npad = nblk * NB
    sc_quant = SC_WORKERS * SC_G
    mpad = -(-(m + BIG) // sc_quant) * sc_quant

    trg_s, src_s = lax.sort((trg, src), num_keys=1, is_stable=False)
    bounds = (jnp.arange(nblk + 1, dtype=jnp.int32) * NB)
    starts = jnp.searchsorted(trg_s, bounds, side="left").astype(jnp.int32)

    pad_m = mpad - m
    src_pad = jnp.concatenate(
        [src_s, (jnp.arange(pad_m, dtype=jnp.int32) % n)])
    trg_pad = jnp.concatenate(
        [trg_s, jnp.full((pad_m,), npad, jnp.int32)]).reshape(1, mpad)

    x_pad = jnp.pad(x, ((0, npad - n), (0, 0)))
    xn_pad = _normalize(x_pad)
    z = _gather_rows(xn_pad, src_pad)
    c = _routing(starts, xn_pad, z, trg_pad, npad)
    return c[:n]


# speedup vs baseline: 2.1791x; 2.1791x over previous
import functools

import jax
import jax.numpy as jnp
from jax import lax
from jax.experimental import pallas as pl
from jax.experimental.pallas import tpu as pltpu
from jax.experimental.pallas import tpu_sc as plsc

KCAP = 4
DD = 32
D = 128
NB = 128
C = 2048
BIGC = 4
BIG = BIGC * C
ROUTE_ITERS = 6
SC_G = 512
SC_WORKERS = 32


def _sel(i, j):
    return (i == j).astype(jnp.float32)


def _mk_consts():
    w1 = _sel(lax.broadcasted_iota(jnp.int32, (D, D), 0) // KCAP,
              lax.broadcasted_iota(jnp.int32, (D, D), 1) % DD)
    g = _sel(lax.broadcasted_iota(jnp.int32, (D, KCAP), 0) // DD,
             lax.broadcasted_iota(jnp.int32, (D, KCAP), 1))
    gt = _sel(lax.broadcasted_iota(jnp.int32, (KCAP, D), 0),
              lax.broadcasted_iota(jnp.int32, (KCAP, D), 1) // DD)
    return w1, g, gt


def _group_normalize(v, g, gt):
    nrm2 = jnp.dot(v * v, g, preferred_element_type=jnp.float32)
    inv = 1.0 / jnp.maximum(jnp.sqrt(nrm2), 1e-12)
    return v * jnp.dot(inv, gt, preferred_element_type=jnp.float32)


def _norm_body(x_ref, o_ref):
    _, g, gt = _mk_consts()
    o_ref[...] = _group_normalize(x_ref[...], g, gt)


def _normalize(x_pad):
    return pl.pallas_call(
        _norm_body,
        out_shape=jax.ShapeDtypeStruct(x_pad.shape, jnp.float32),
    )(x_pad)


def _gather_rows(table, idx):
    mpad = idx.shape[0]
    per_w = mpad // SC_WORKERS
    nchunk = per_w // SC_G
    mesh = plsc.VectorSubcoreMesh(core_axis_name="c", subcore_axis_name="s")

    @functools.partial(
        pl.kernel,
        mesh=mesh,
        out_type=jax.ShapeDtypeStruct((mpad, D), jnp.float32),
        scratch_types=[
            pltpu.VMEM((SC_G,), jnp.int32),
            pltpu.VMEM((SC_G, D), jnp.float32),
            pltpu.SemaphoreType.DMA,
        ],
    )
    def sc_gather(table_hbm, idx_hbm, out_hbm, idx_v, rows_v, sem):
        wid = lax.axis_index("s") * 2 + lax.axis_index("c")
        base = wid * per_w

        @pl.loop(0, nchunk)
        def _(i):
            off = base + i * SC_G
            pltpu.sync_copy(idx_hbm.at[pl.ds(off, SC_G)], idx_v)
            pltpu.async_copy(table_hbm.at[idx_v], rows_v, sem).wait()
            pltpu.sync_copy(rows_v, out_hbm.at[pl.ds(off, SC_G)])

    return sc_gather(table, idx)


def _gsum(vt):
    return jnp.concatenate(
        [jnp.sum(vt[k * DD:(k + 1) * DD, :], axis=0, keepdims=True)
         for k in range(KCAP)], axis=0)


def _gexpand(v4):
    return jnp.concatenate(
        [jnp.broadcast_to(v4[k:k + 1, :], (DD, v4.shape[1]))
         for k in range(KCAP)], axis=0)


def _routing_body(starts_ref, xn_ref, z_hbm, trg_hbm, out_ref,
                  z_land, z_buf, trg_buf, dma_sem):
    b = pl.program_id(0)
    start = pl.multiple_of((starts_ref[b] // NB) * NB, NB)
    n_e = starts_ref[b + 1] - start
    nch = lax.div(n_e + (C - 1), C)
    npair = lax.div(nch + 1, 2)
    fits = n_e <= BIG

    w1t = _sel(lax.broadcasted_iota(jnp.int32, (D, D), 0) % DD,
               lax.broadcasted_iota(jnp.int32, (D, D), 1) // KCAP)
    base_n = b * NB
    iota_n = lax.broadcasted_iota(jnp.int32, (NB, C), 0) + base_n

    x_blk_t = jnp.transpose(xn_ref[...])
    c_t = x_blk_t

    def load_transpose(i, slot, li):
        cp_z = pltpu.make_async_copy(
            z_hbm.at[pl.ds(start + i * C, C), :], z_land.at[li], dma_sem)
        cp_t = pltpu.make_async_copy(
            trg_hbm.at[:, pl.ds(start + i * C, C)],
            trg_buf.at[slot], dma_sem)
        cp_z.start()
        cp_t.start()
        return cp_z, cp_t

    for t in range(ROUTE_ITERS):
        cst_t16 = lax.dot_general(
            w1t, c_t, (((1,), (0,)), ((), ())),
            preferred_element_type=jnp.float32).astype(jnp.bfloat16)

        def chunk_math(slot, acc_t):
            z_t = z_buf[slot]
            trg_c = trg_buf[slot]
            ot = _sel(iota_n, trg_c)
            ot16 = ot.astype(jnp.bfloat16)
            csg_t = lax.dot_general(cst_t16, ot16, (((1,), (0,)), ((), ())),
                                    preferred_element_type=jnp.float32)
            p4 = _gsum(z_t * csg_t)
            e4 = jnp.exp(p4)
            p4n = e4 / jnp.sum(e4, axis=0, keepdims=True)
            w_t = (_gexpand(p4n) * z_t).astype(jnp.bfloat16)
            return acc_t + lax.dot_general(
                w_t, ot16, (((1,), (1,)), ((), ())),
                preferred_element_type=jnp.float32)

        def pair_body(j, acc_t, first):
            i0 = 2 * j
            s0 = jnp.where(fits, i0, 0)
            s1 = jnp.where(fits, i0 + 1, 1)

            def do_dma():
                cps = load_transpose(i0, s0, 0) + load_transpose(i0 + 1, s1, 1)
                for cp in cps:
                    cp.wait()
                z_buf[s0] = jnp.transpose(z_land[0])
                z_buf[s1] = jnp.transpose(z_land[1])

            if first:
                do_dma()
            else:
                pl.when(jnp.logical_not(fits))(do_dma)
            acc_t = chunk_math(s0, acc_t)
            return chunk_math(s1, acc_t)

        acc0 = jnp.zeros((D, NB), jnp.float32)
        acc_t = lax.fori_loop(0, npair,
                              lambda j, a: pair_body(j, a, t == 0), acc0)
        c_t = x_blk_t + acc_t
        if t < ROUTE_ITERS - 1:
            nrm2 = _gsum(c_t * c_t)
            inv = 1.0 / jnp.maximum(jnp.sqrt(nrm2), 1e-12)
            c_t = c_t * _gexpand(inv)

    out_ref[...] = jnp.transpose(c_t)


def _routing(starts, xn_pad, z, trg_pad2d, npad):
    grid_spec = pltpu.PrefetchScalarGridSpec(
        num_scalar_prefetch=1,
        grid=(npad // NB,),
        in_specs=[
            pl.BlockSpec((NB, D), lambda b, s: (b, 0)),
            pl.BlockSpec(memory_space=pl.ANY),
            pl.BlockSpec(memory_space=pl.ANY),
        ],
        out_specs=pl.BlockSpec((NB, D), lambda b, s: (b, 0)),
        scratch_shapes=[
            pltpu.VMEM((2, C, D), jnp.float32),
            pltpu.VMEM((BIGC, D, C), jnp.float32),
            pltpu.VMEM((BIGC, 1, C), jnp.int32),
            pltpu.SemaphoreType.DMA,
        ],
    )
    return pl.pallas_call(
        _routing_body,
        grid_spec=grid_spec,
        out_shape=jax.ShapeDtypeStruct((npad, D), jnp.float32),
    )(starts, xn_pad, z, trg_pad2d)


def kernel(x, edge_index):
    n, d = x.shape
    assert d == D
    src = edge_index[0]
    trg = edge_index[1]
    m = src.shape[0]

    nblk = -(-n // NB)
    npad = nblk * NB
    sc_quant = SC_WORKERS * SC_G
    mpad = -(-(m + BIG) // sc_quant) * sc_quant

    trg_s, src_s = lax.sort((trg, src), num_keys=1, is_stable=False)
    bounds = (jnp.arange(nblk + 1, dtype=jnp.int32) * NB)
    starts = jnp.searchsorted(trg_s, bounds, side="left").astype(jnp.int32)

    pad_m = mpad - m
    src_pad = jnp.concatenate(
        [src_s, (jnp.arange(pad_m, dtype=jnp.int32) % n)])
    trg_pad = jnp.concatenate(
        [trg_s, jnp.full((pad_m,), npad, jnp.int32)]).reshape(1, mpad)

    x_pad = jnp.pad(x, ((0, npad - n), (0, 0)))
    xn_pad = _normalize(x_pad)
    z = _gather_rows(xn_pad, src_pad)
    c = _routing(starts, xn_pad, z, trg_pad, npad)
    return c[:n]


# speedup vs baseline: 2.1796x; 1.0002x over previous
import functools

import jax
import jax.numpy as jnp
from jax import lax
from jax.experimental import pallas as pl
from jax.experimental.pallas import tpu as pltpu
from jax.experimental.pallas import tpu_sc as plsc

KCAP = 4
DD = 32
D = 128
NB = 128
C = 2048
BIGC = 4
BIG = BIGC * C
ROUTE_ITERS = 6
SC_G = 512
SC_WORKERS = 32


def _sel(i, j):
    return (i == j).astype(jnp.float32)


def _mk_consts():
    w1 = _sel(lax.broadcasted_iota(jnp.int32, (D, D), 0) // KCAP,
              lax.broadcasted_iota(jnp.int32, (D, D), 1) % DD)
    g = _sel(lax.broadcasted_iota(jnp.int32, (D, KCAP), 0) // DD,
             lax.broadcasted_iota(jnp.int32, (D, KCAP), 1))
    gt = _sel(lax.broadcasted_iota(jnp.int32, (KCAP, D), 0),
              lax.broadcasted_iota(jnp.int32, (KCAP, D), 1) // DD)
    return w1, g, gt


def _group_normalize(v, g, gt):
    nrm2 = jnp.dot(v * v, g, preferred_element_type=jnp.float32)
    inv = 1.0 / jnp.maximum(jnp.sqrt(nrm2), 1e-12)
    return v * jnp.dot(inv, gt, preferred_element_type=jnp.float32)


def _norm_body(x_ref, o_ref):
    _, g, gt = _mk_consts()
    o_ref[...] = _group_normalize(x_ref[...], g, gt)


def _normalize(x_pad):
    return pl.pallas_call(
        _norm_body,
        out_shape=jax.ShapeDtypeStruct(x_pad.shape, jnp.float32),
    )(x_pad)


def _gather_rows(table, idx):
    mpad = idx.shape[0]
    per_w = mpad // SC_WORKERS
    nchunk = per_w // SC_G
    mesh = plsc.VectorSubcoreMesh(core_axis_name="c", subcore_axis_name="s")

    @functools.partial(
        pl.kernel,
        mesh=mesh,
        out_type=jax.ShapeDtypeStruct((mpad, D), jnp.float32),
        scratch_types=[
            pltpu.VMEM((SC_G,), jnp.int32),
            pltpu.VMEM((SC_G, D), jnp.float32),
            pltpu.SemaphoreType.DMA,
        ],
    )
    def sc_gather(table_hbm, idx_hbm, out_hbm, idx_v, rows_v, sem):
        wid = lax.axis_index("s") * 2 + lax.axis_index("c")
        base = wid * per_w

        @pl.loop(0, nchunk)
        def _(i):
            off = base + i * SC_G
            pltpu.sync_copy(idx_hbm.at[pl.ds(off, SC_G)], idx_v)
            pltpu.async_copy(table_hbm.at[idx_v], rows_v, sem).wait()
            pltpu.sync_copy(rows_v, out_hbm.at[pl.ds(off, SC_G)])

    return sc_gather(table, idx)


def _gsum(vt):
    return jnp.concatenate(
        [jnp.sum(vt[k * DD:(k + 1) * DD, :], axis=0, keepdims=True)
         for k in range(KCAP)], axis=0)


def _gexpand(v4):
    return jnp.concatenate(
        [jnp.broadcast_to(v4[k:k + 1, :], (DD, v4.shape[1]))
         for k in range(KCAP)], axis=0)


def _routing_body(starts_ref, xn_ref, z_hbm, trg_hbm, out_ref,
                  z_land, z_buf, trg_buf, dma_sem):
    b = pl.program_id(0)
    start = pl.multiple_of((starts_ref[b] // NB) * NB, NB)
    n_e = starts_ref[b + 1] - start
    nch = lax.div(n_e + (C - 1), C)
    npair = lax.div(nch + 1, 2)
    fits = n_e <= BIG

    w1t = _sel(lax.broadcasted_iota(jnp.int32, (D, D), 0) % DD,
               lax.broadcasted_iota(jnp.int32, (D, D), 1) // KCAP)
    base_n = b * NB
    iota_n = lax.broadcasted_iota(jnp.int32, (NB, C), 0) + base_n

    x_blk_t = jnp.transpose(xn_ref[...])
    c_t = x_blk_t

    def load_transpose(i, slot, li):
        cp_z = pltpu.make_async_copy(
            z_hbm.at[pl.ds(start + i * C, C), :], z_land.at[li], dma_sem)
        cp_t = pltpu.make_async_copy(
            trg_hbm.at[:, pl.ds(start + i * C, C)],
            trg_buf.at[slot], dma_sem)
        cp_z.start()
        cp_t.start()
        return cp_z, cp_t

    for t in range(ROUTE_ITERS):
        cst_t16 = lax.dot_general(
            w1t, c_t, (((1,), (0,)), ((), ())),
            preferred_element_type=jnp.float32).astype(jnp.bfloat16)

        def chunk_math(slot, acc_t):
            z_t = z_buf[slot]
            trg_c = trg_buf[slot]
            ot = _sel(iota_n, trg_c)
            ot16 = ot.astype(jnp.bfloat16)
            csg_t = lax.dot_general(cst_t16, ot16, (((1,), (0,)), ((), ())),
                                    preferred_element_type=jnp.float32)
            p4 = _gsum(z_t * csg_t)
            e4 = jnp.exp(p4)
            p4n = e4 / jnp.sum(e4, axis=0, keepdims=True)
            w_t = (_gexpand(p4n) * z_t).astype(jnp.bfloat16)
            return acc_t + lax.dot_general(
                w_t, ot16, (((1,), (1,)), ((), ())),
                preferred_element_type=jnp.float32)

        def pair_body(j, acc_t, first):
            i0 = 2 * j
            s0 = jnp.where(fits, i0, 0)
            s1 = jnp.where(fits, i0 + 1, 1)

            def do_dma():
                cps = load_transpose(i0, s0, 0) + load_transpose(i0 + 1, s1, 1)
                for cp in cps:
                    cp.wait()
                z_buf[s0] = jnp.transpose(z_land[0])
                z_buf[s1] = jnp.transpose(z_land[1])

            if first:
                do_dma()
            else:
                pl.when(jnp.logical_not(fits))(do_dma)
            acc_t = chunk_math(s0, acc_t)
            return chunk_math(s1, acc_t)

        acc0 = jnp.zeros((D, NB), jnp.float32)
        acc_t = lax.fori_loop(0, npair,
                              lambda j, a: pair_body(j, a, t == 0), acc0)
        c_t = x_blk_t + acc_t
        if t < ROUTE_ITERS - 1:
            nrm2 = _gsum(c_t * c_t)
            inv = 1.0 / jnp.maximum(jnp.sqrt(nrm2), 1e-12)
            c_t = c_t * _gexpand(inv)

    out_ref[...] = jnp.transpose(c_t)


def _routing(starts, xn_pad, z, trg_pad2d, npad):
    grid_spec = pltpu.PrefetchScalarGridSpec(
        num_scalar_prefetch=1,
        grid=(npad // NB,),
        in_specs=[
            pl.BlockSpec((NB, D), lambda b, s: (b, 0)),
            pl.BlockSpec(memory_space=pl.ANY),
            pl.BlockSpec(memory_space=pl.ANY),
        ],
        out_specs=pl.BlockSpec((NB, D), lambda b, s: (b, 0)),
        scratch_shapes=[
            pltpu.VMEM((2, C, D), jnp.float32),
            pltpu.VMEM((BIGC, D, C), jnp.float32),
            pltpu.VMEM((BIGC, 1, C), jnp.int32),
            pltpu.SemaphoreType.DMA,
        ],
    )
    return pl.pallas_call(
        _routing_body,
        grid_spec=grid_spec,
        out_shape=jax.ShapeDtypeStruct((npad, D), jnp.float32),
        compiler_params=pltpu.CompilerParams(
            dimension_semantics=("parallel",)),
    )(starts, xn_pad, z, trg_pad2d)


def kernel(x, edge_index):
    n, d = x.shape
    assert d == D
    src = edge_index[0]
    trg = edge_index[1]
    m = src.shape[0]

    nblk = -(-n // NB)
    npad = nblk * NB
    sc_quant = SC_WORKERS * SC_G
    mpad = -(-(m + BIG) // sc_quant) * sc_quant

    trg_s, src_s = lax.sort((trg, src), num_keys=1, is_stable=False)
    bounds = (jnp.arange(nblk + 1, dtype=jnp.int32) * NB)
    starts = jnp.searchsorted(trg_s, bounds, side="left").astype(jnp.int32)

    pad_m = mpad - m
    src_pad = jnp.concatenate(
        [src_s, (jnp.arange(pad_m, dtype=jnp.int32) % n)])
    trg_pad = jnp.concatenate(
        [trg_s, jnp.full((pad_m,), npad, jnp.int32)]).reshape(1, mpad)

    x_pad = jnp.pad(x, ((0, npad - n), (0, 0)))
    xn_pad = _normalize(x_pad)
    z = _gather_rows(xn_pad, src_pad)
    c = _routing(starts, xn_pad, z, trg_pad, npad)
    return c[:n]


# speedup vs baseline: 2.1835x; 1.0018x over previous
import functools

import jax
import jax.numpy as jnp
from jax import lax
from jax.experimental import pallas as pl
from jax.experimental.pallas import tpu as pltpu
from jax.experimental.pallas import tpu_sc as plsc
from jax.experimental.compute_on import compute_on

KCAP = 4
DD = 32
D = 128
NB = 128
C = 2048
BIGC = 4
BIG = BIGC * C
ROUTE_ITERS = 6
SC_G = 512
SC_WORKERS = 32


def _sel(i, j):
    return (i == j).astype(jnp.float32)


def _mk_consts():
    w1 = _sel(lax.broadcasted_iota(jnp.int32, (D, D), 0) // KCAP,
              lax.broadcasted_iota(jnp.int32, (D, D), 1) % DD)
    g = _sel(lax.broadcasted_iota(jnp.int32, (D, KCAP), 0) // DD,
             lax.broadcasted_iota(jnp.int32, (D, KCAP), 1))
    gt = _sel(lax.broadcasted_iota(jnp.int32, (KCAP, D), 0),
              lax.broadcasted_iota(jnp.int32, (KCAP, D), 1) // DD)
    return w1, g, gt


def _group_normalize(v, g, gt):
    nrm2 = jnp.dot(v * v, g, preferred_element_type=jnp.float32)
    inv = 1.0 / jnp.maximum(jnp.sqrt(nrm2), 1e-12)
    return v * jnp.dot(inv, gt, preferred_element_type=jnp.float32)


def _norm_body(x_ref, o_ref):
    _, g, gt = _mk_consts()
    o_ref[...] = _group_normalize(x_ref[...], g, gt)


def _normalize(x_pad):
    return pl.pallas_call(
        _norm_body,
        out_shape=jax.ShapeDtypeStruct(x_pad.shape, jnp.float32),
    )(x_pad)


def _gather_rows(table, idx):
    mpad = idx.shape[0]
    per_w = mpad // SC_WORKERS
    nchunk = per_w // SC_G
    mesh = plsc.VectorSubcoreMesh(core_axis_name="c", subcore_axis_name="s")

    @functools.partial(
        pl.kernel,
        mesh=mesh,
        out_type=jax.ShapeDtypeStruct((mpad, D), jnp.float32),
        scratch_types=[
            pltpu.VMEM((SC_G,), jnp.int32),
            pltpu.VMEM((SC_G, D), jnp.float32),
            pltpu.SemaphoreType.DMA,
        ],
    )
    def sc_gather(table_hbm, idx_hbm, out_hbm, idx_v, rows_v, sem):
        wid = lax.axis_index("s") * 2 + lax.axis_index("c")
        base = wid * per_w

        @pl.loop(0, nchunk)
        def _(i):
            off = base + i * SC_G
            pltpu.sync_copy(idx_hbm.at[pl.ds(off, SC_G)], idx_v)
            pltpu.async_copy(table_hbm.at[idx_v], rows_v, sem).wait()
            pltpu.sync_copy(rows_v, out_hbm.at[pl.ds(off, SC_G)])

    return sc_gather(table, idx)


def _gsum(vt):
    return jnp.concatenate(
        [jnp.sum(vt[k * DD:(k + 1) * DD, :], axis=0, keepdims=True)
         for k in range(KCAP)], axis=0)


def _gexpand(v4):
    return jnp.concatenate(
        [jnp.broadcast_to(v4[k:k + 1, :], (DD, v4.shape[1]))
         for k in range(KCAP)], axis=0)


def _routing_body(starts_ref, xn_ref, z_hbm, trg_hbm, out_ref,
                  z_land, z_buf, trg_buf, dma_sem):
    b = pl.program_id(0)
    start = pl.multiple_of((starts_ref[b] // NB) * NB, NB)
    n_e = starts_ref[b + 1] - start
    nch = lax.div(n_e + (C - 1), C)
    npair = lax.div(nch + 1, 2)
    fits = n_e <= BIG

    w1t = _sel(lax.broadcasted_iota(jnp.int32, (D, D), 0) % DD,
               lax.broadcasted_iota(jnp.int32, (D, D), 1) // KCAP)
    base_n = b * NB
    iota_n = lax.broadcasted_iota(jnp.int32, (NB, C), 0) + base_n

    x_blk_t = jnp.transpose(xn_ref[...])
    c_t = x_blk_t

    def load_transpose(i, slot, li):
        cp_z = pltpu.make_async_copy(
            z_hbm.at[pl.ds(start + i * C, C), :], z_land.at[li], dma_sem)
        cp_t = pltpu.make_async_copy(
            trg_hbm.at[:, pl.ds(start + i * C, C)],
            trg_buf.at[slot], dma_sem)
        cp_z.start()
        cp_t.start()
        return cp_z, cp_t

    for t in range(ROUTE_ITERS):
        cst_t16 = lax.dot_general(
            w1t, c_t, (((1,), (0,)), ((), ())),
            preferred_element_type=jnp.float32).astype(jnp.bfloat16)

        def chunk_math(slot, acc_t):
            z_t = z_buf[slot]
            trg_c = trg_buf[slot]
            ot = _sel(iota_n, trg_c)
            ot16 = ot.astype(jnp.bfloat16)
            csg_t = lax.dot_general(cst_t16, ot16, (((1,), (0,)), ((), ())),
                                    preferred_element_type=jnp.float32)
            p4 = _gsum(z_t * csg_t)
            e4 = jnp.exp(p4)
            p4n = e4 / jnp.sum(e4, axis=0, keepdims=True)
            w_t = (_gexpand(p4n) * z_t).astype(jnp.bfloat16)
            return acc_t + lax.dot_general(
                w_t, ot16, (((1,), (1,)), ((), ())),
                preferred_element_type=jnp.float32)

        def pair_body(j, acc_t, first):
            i0 = 2 * j
            s0 = jnp.where(fits, i0, 0)
            s1 = jnp.where(fits, i0 + 1, 1)

            def do_dma():
                cps = load_transpose(i0, s0, 0) + load_transpose(i0 + 1, s1, 1)
                for cp in cps:
                    cp.wait()
                z_buf[s0] = jnp.transpose(z_land[0])
                z_buf[s1] = jnp.transpose(z_land[1])

            if first:
                do_dma()
            else:
                pl.when(jnp.logical_not(fits))(do_dma)
            acc_t = chunk_math(s0, acc_t)
            return chunk_math(s1, acc_t)

        acc0 = jnp.zeros((D, NB), jnp.float32)
        acc_t = lax.fori_loop(0, npair,
                              lambda j, a: pair_body(j, a, t == 0), acc0)
        c_t = x_blk_t + acc_t
        if t < ROUTE_ITERS - 1:
            nrm2 = _gsum(c_t * c_t)
            inv = 1.0 / jnp.maximum(jnp.sqrt(nrm2), 1e-12)
            c_t = c_t * _gexpand(inv)

    out_ref[...] = jnp.transpose(c_t)


def _routing(starts, xn_pad, z, trg_pad2d, npad):
    grid_spec = pltpu.PrefetchScalarGridSpec(
        num_scalar_prefetch=1,
        grid=(npad // NB,),
        in_specs=[
            pl.BlockSpec((NB, D), lambda b, s: (b, 0)),
            pl.BlockSpec(memory_space=pl.ANY),
            pl.BlockSpec(memory_space=pl.ANY),
        ],
        out_specs=pl.BlockSpec((NB, D), lambda b, s: (b, 0)),
        scratch_shapes=[
            pltpu.VMEM((2, C, D), jnp.float32),
            pltpu.VMEM((BIGC, D, C), jnp.float32),
            pltpu.VMEM((BIGC, 1, C), jnp.int32),
            pltpu.SemaphoreType.DMA,
        ],
    )
    return pl.pallas_call(
        _routing_body,
        grid_spec=grid_spec,
        out_shape=jax.ShapeDtypeStruct((npad, D), jnp.float32),
        compiler_params=pltpu.CompilerParams(
            dimension_semantics=("parallel",)),
    )(starts, xn_pad, z, trg_pad2d)


def kernel(x, edge_index):
    n, d = x.shape
    assert d == D
    src = edge_index[0]
    trg = edge_index[1]
    m = src.shape[0]

    nblk = -(-n // NB)
    npad = nblk * NB
    sc_quant = SC_WORKERS * SC_G
    mpad = -(-(m + BIG) // sc_quant) * sc_quant

    with compute_on("tpu_sparsecore"):
        trg_s, src_s = lax.sort((trg, src), num_keys=1, is_stable=False)
    bounds = (jnp.arange(nblk + 1, dtype=jnp.int32) * NB)
    starts = jnp.searchsorted(trg_s, bounds, side="left").astype(jnp.int32)

    pad_m = mpad - m
    src_pad = jnp.concatenate(
        [src_s, (jnp.arange(pad_m, dtype=jnp.int32) % n)])
    trg_pad = jnp.concatenate(
        [trg_s, jnp.full((pad_m,), npad, jnp.int32)]).reshape(1, mpad)

    x_pad = jnp.pad(x, ((0, npad - n), (0, 0)))
    xn_pad = _normalize(x_pad)
    z = _gather_rows(xn_pad, src_pad)
    c = _routing(starts, xn_pad, z, trg_pad, npad)
    return c[:n]


# speedup vs baseline: 2.1853x; 1.0008x over previous
import functools

import jax
import jax.numpy as jnp
from jax import lax
from jax.experimental import pallas as pl
from jax.experimental.pallas import tpu as pltpu
from jax.experimental.pallas import tpu_sc as plsc

KCAP = 4
DD = 32
D = 128
NB = 128
C = 2048
BIGC = 4
BIG = BIGC * C
ROUTE_ITERS = 6
SC_G = 512
SC_WORKERS = 32


def _sel(i, j):
    return (i == j).astype(jnp.float32)


def _mk_consts():
    w1 = _sel(lax.broadcasted_iota(jnp.int32, (D, D), 0) // KCAP,
              lax.broadcasted_iota(jnp.int32, (D, D), 1) % DD)
    g = _sel(lax.broadcasted_iota(jnp.int32, (D, KCAP), 0) // DD,
             lax.broadcasted_iota(jnp.int32, (D, KCAP), 1))
    gt = _sel(lax.broadcasted_iota(jnp.int32, (KCAP, D), 0),
              lax.broadcasted_iota(jnp.int32, (KCAP, D), 1) // DD)
    return w1, g, gt


def _group_normalize(v, g, gt):
    nrm2 = jnp.dot(v * v, g, preferred_element_type=jnp.float32)
    inv = 1.0 / jnp.maximum(jnp.sqrt(nrm2), 1e-12)
    return v * jnp.dot(inv, gt, preferred_element_type=jnp.float32)


def _norm_body(x_ref, o_ref):
    _, g, gt = _mk_consts()
    o_ref[...] = _group_normalize(x_ref[...], g, gt)


def _normalize(x_pad):
    return pl.pallas_call(
        _norm_body,
        out_shape=jax.ShapeDtypeStruct(x_pad.shape, jnp.float32),
    )(x_pad)


def _gather_rows(table, idx):
    mpad = idx.shape[0]
    per_w = mpad // SC_WORKERS
    nchunk = per_w // SC_G
    mesh = plsc.VectorSubcoreMesh(core_axis_name="c", subcore_axis_name="s")

    @functools.partial(
        pl.kernel,
        mesh=mesh,
        out_type=jax.ShapeDtypeStruct((mpad, D), jnp.float32),
        scratch_types=[
            pltpu.VMEM((SC_G,), jnp.int32),
            pltpu.VMEM((SC_G, D), jnp.float32),
            pltpu.SemaphoreType.DMA,
        ],
    )
    def sc_gather(table_hbm, idx_hbm, out_hbm, idx_v, rows_v, sem):
        wid = lax.axis_index("s") * 2 + lax.axis_index("c")
        base = wid * per_w

        @pl.loop(0, nchunk)
        def _(i):
            off = base + i * SC_G
            pltpu.sync_copy(idx_hbm.at[pl.ds(off, SC_G)], idx_v)
            pltpu.async_copy(table_hbm.at[idx_v], rows_v, sem).wait()
            pltpu.sync_copy(rows_v, out_hbm.at[pl.ds(off, SC_G)])

    return sc_gather(table, idx)


def _gsum(vt):
    return jnp.concatenate(
        [jnp.sum(vt[k * DD:(k + 1) * DD, :], axis=0, keepdims=True)
         for k in range(KCAP)], axis=0)


def _gexpand(v4):
    return jnp.concatenate(
        [jnp.broadcast_to(v4[k:k + 1, :], (DD, v4.shape[1]))
         for k in range(KCAP)], axis=0)


def _routing_body(starts_ref, xn_ref, z_hbm, trg_hbm, out_ref,
                  z_land, z_buf, z16_buf, ot16_buf, trg_buf, dma_sem):
    b = pl.program_id(0)
    start = pl.multiple_of((starts_ref[b] // NB) * NB, NB)
    n_e = starts_ref[b + 1] - start
    nch = lax.div(n_e + (C - 1), C)
    npair = lax.div(nch + 1, 2)
    fits = n_e <= BIG

    w1t = _sel(lax.broadcasted_iota(jnp.int32, (D, D), 0) % DD,
               lax.broadcasted_iota(jnp.int32, (D, D), 1) // KCAP)
    base_n = b * NB
    iota_n = lax.broadcasted_iota(jnp.int32, (NB, C), 0) + base_n

    x_blk_t = jnp.transpose(xn_ref[...])
    c_t = x_blk_t

    def load_transpose(i, slot, li):
        cp_z = pltpu.make_async_copy(
            z_hbm.at[pl.ds(start + i * C, C), :], z_land.at[li], dma_sem)
        cp_t = pltpu.make_async_copy(
            trg_hbm.at[:, pl.ds(start + i * C, C)],
            trg_buf.at[slot], dma_sem)
        cp_z.start()
        cp_t.start()
        return cp_z, cp_t

    for t in range(ROUTE_ITERS):
        cst_t16 = lax.dot_general(
            w1t, c_t, (((1,), (0,)), ((), ())),
            preferred_element_type=jnp.float32).astype(jnp.bfloat16)

        def chunk_math(slot, acc_t):
            z_t = z_buf[slot]
            z16 = z16_buf[slot]
            ot16 = ot16_buf[slot]
            csg_t = lax.dot_general(cst_t16, ot16, (((1,), (0,)), ((), ())),
                                    preferred_element_type=jnp.float32)
            p4 = _gsum(z_t * csg_t)
            e4 = jnp.exp(p4)
            p4n = (e4 / jnp.sum(e4, axis=0, keepdims=True)).astype(jnp.bfloat16)
            w16 = _gexpand(p4n) * z16
            return acc_t + lax.dot_general(
                w16, ot16, (((1,), (1,)), ((), ())),
                preferred_element_type=jnp.float32)

        def pair_body(j, acc_t, first):
            i0 = 2 * j
            s0 = jnp.where(fits, i0, 0)
            s1 = jnp.where(fits, i0 + 1, 1)

            def do_dma():
                cps = load_transpose(i0, s0, 0) + load_transpose(i0 + 1, s1, 1)
                for cp in cps:
                    cp.wait()
                for sl, li in ((s0, 0), (s1, 1)):
                    z_t = jnp.transpose(z_land[li])
                    z_buf[sl] = z_t
                    z16_buf[sl] = z_t.astype(jnp.bfloat16)
                    ot16_buf[sl] = _sel(iota_n, trg_buf[sl]).astype(
                        jnp.bfloat16)

            if first:
                do_dma()
            else:
                pl.when(jnp.logical_not(fits))(do_dma)
            acc_t = chunk_math(s0, acc_t)
            return chunk_math(s1, acc_t)

        acc0 = jnp.zeros((D, NB), jnp.float32)
        acc_t = lax.fori_loop(0, npair,
                              lambda j, a: pair_body(j, a, t == 0), acc0)
        c_t = x_blk_t + acc_t
        if t < ROUTE_ITERS - 1:
            nrm2 = _gsum(c_t * c_t)
            inv = 1.0 / jnp.maximum(jnp.sqrt(nrm2), 1e-12)
            c_t = c_t * _gexpand(inv)

    out_ref[...] = jnp.transpose(c_t)


def _routing(starts, xn_pad, z, trg_pad2d, npad):
    grid_spec = pltpu.PrefetchScalarGridSpec(
        num_scalar_prefetch=1,
        grid=(npad // NB,),
        in_specs=[
            pl.BlockSpec((NB, D), lambda b, s: (b, 0)),
            pl.BlockSpec(memory_space=pl.ANY),
            pl.BlockSpec(memory_space=pl.ANY),
        ],
        out_specs=pl.BlockSpec((NB, D), lambda b, s: (b, 0)),
        scratch_shapes=[
            pltpu.VMEM((2, C, D), jnp.float32),
            pltpu.VMEM((BIGC, D, C), jnp.float32),
            pltpu.VMEM((BIGC, D, C), jnp.bfloat16),
            pltpu.VMEM((BIGC, NB, C), jnp.bfloat16),
            pltpu.VMEM((BIGC, 1, C), jnp.int32),
            pltpu.SemaphoreType.DMA,
        ],
    )
    return pl.pallas_call(
        _routing_body,
        grid_spec=grid_spec,
        out_shape=jax.ShapeDtypeStruct((npad, D), jnp.float32),
        compiler_params=pltpu.CompilerParams(
            dimension_semantics=("parallel",)),
    )(starts, xn_pad, z, trg_pad2d)


def kernel(x, edge_index):
    n, d = x.shape
    assert d == D
    src = edge_index[0]
    trg = edge_index[1]
    m = src.shape[0]

    nblk = -(-n // NB)
    npad = nblk * NB
    sc_quant = SC_WORKERS * SC_G
    mpad = -(-(m + BIG) // sc_quant) * sc_quant

    trg_s, src_s = lax.sort((trg, src), num_keys=1, is_stable=False)
    bounds = (jnp.arange(nblk + 1, dtype=jnp.int32) * NB)
    starts = jnp.searchsorted(trg_s, bounds, side="left").astype(jnp.int32)

    pad_m = mpad - m
    src_pad = jnp.concatenate(
        [src_s, (jnp.arange(pad_m, dtype=jnp.int32) % n)])
    trg_pad = jnp.concatenate(
        [trg_s, jnp.full((pad_m,), npad, jnp.int32)]).reshape(1, mpad)

    x_pad = jnp.pad(x, ((0, npad - n), (0, 0)))
    xn_pad = _normalize(x_pad)
    z = _gather_rows(xn_pad, src_pad)
    c = _routing(starts, xn_pad, z, trg_pad, npad)
    return c[:n]


# speedup vs baseline: 2.5346x; 1.1599x over previous
import functools

import jax
import jax.numpy as jnp
from jax import lax
from jax.experimental import pallas as pl
from jax.experimental.pallas import tpu as pltpu
from jax.experimental.pallas import tpu_sc as plsc

KCAP = 4
DD = 32
D = 128
NB = 128
C = 2048
BIGC = 4
BIG = BIGC * C
ROUTE_ITERS = 6
SC_G = 512
SC_WORKERS = 32


def _sel(i, j):
    return (i == j).astype(jnp.float32)


def _mk_consts():
    w1 = _sel(lax.broadcasted_iota(jnp.int32, (D, D), 0) // KCAP,
              lax.broadcasted_iota(jnp.int32, (D, D), 1) % DD)
    g = _sel(lax.broadcasted_iota(jnp.int32, (D, KCAP), 0) // DD,
             lax.broadcasted_iota(jnp.int32, (D, KCAP), 1))
    gt = _sel(lax.broadcasted_iota(jnp.int32, (KCAP, D), 0),
              lax.broadcasted_iota(jnp.int32, (KCAP, D), 1) // DD)
    return w1, g, gt


def _group_normalize(v, g, gt):
    nrm2 = jnp.dot(v * v, g, preferred_element_type=jnp.float32)
    inv = 1.0 / jnp.maximum(jnp.sqrt(nrm2), 1e-12)
    return v * jnp.dot(inv, gt, preferred_element_type=jnp.float32)


def _norm_body(x_ref, o_ref):
    _, g, gt = _mk_consts()
    o_ref[...] = _group_normalize(x_ref[...], g, gt)


def _normalize(x_pad):
    return pl.pallas_call(
        _norm_body,
        out_shape=jax.ShapeDtypeStruct(x_pad.shape, jnp.float32),
    )(x_pad)


def _gather_rows(table, idx):
    mpad = idx.shape[0]
    per_w = mpad // SC_WORKERS
    nchunk = per_w // SC_G
    mesh = plsc.VectorSubcoreMesh(core_axis_name="c", subcore_axis_name="s")

    @functools.partial(
        pl.kernel,
        mesh=mesh,
        out_type=jax.ShapeDtypeStruct((mpad, D), jnp.float32),
        scratch_types=[
            pltpu.VMEM((SC_G,), jnp.int32),
            pltpu.VMEM((SC_G, D), jnp.float32),
            pltpu.SemaphoreType.DMA,
        ],
    )
    def sc_gather(table_hbm, idx_hbm, out_hbm, idx_v, rows_v, sem):
        wid = lax.axis_index("s") * 2 + lax.axis_index("c")
        base = wid * per_w

        @pl.loop(0, nchunk)
        def _(i):
            off = base + i * SC_G
            pltpu.sync_copy(idx_hbm.at[pl.ds(off, SC_G)], idx_v)
            pltpu.async_copy(table_hbm.at[idx_v], rows_v, sem).wait()
            pltpu.sync_copy(rows_v, out_hbm.at[pl.ds(off, SC_G)])

    return sc_gather(table, idx)


def _gsum(vt):
    return jnp.concatenate(
        [jnp.sum(vt[k * DD:(k + 1) * DD, :], axis=0, keepdims=True)
         for k in range(KCAP)], axis=0)


def _gexpand(v4):
    return jnp.concatenate(
        [jnp.broadcast_to(v4[k:k + 1, :], (DD, v4.shape[1]))
         for k in range(KCAP)], axis=0)


def _routing_body(starts_ref, xn_ref, z_hbm, trg_hbm, out_ref,
                  z_land, trg_land, s_land, strg_land,
                  z_buf, z16_buf, ot16_buf, dma_sem, pre_sem):
    b = pl.program_id(0)
    nblk = pl.num_programs(0)
    start = pl.multiple_of((starts_ref[b] // NB) * NB, NB)
    n_e = starts_ref[b + 1] - start
    nch = lax.div(n_e + (C - 1), C)
    npair = lax.div(nch + 1, 2)
    fits = n_e <= BIG

    w1t = _sel(lax.broadcasted_iota(jnp.int32, (D, D), 0) % DD,
               lax.broadcasted_iota(jnp.int32, (D, D), 1) // KCAP)
    base_n = b * NB
    iota_n = lax.broadcasted_iota(jnp.int32, (NB, C), 0) + base_n

    def prefetch_copies(bb, pp):
        st = pl.multiple_of((starts_ref[bb] // NB) * NB, NB)
        cps = []
        for i in range(BIGC):
            cps.append(pltpu.make_async_copy(
                z_hbm.at[pl.ds(st + i * C, C), :], z_land.at[pp, i], pre_sem))
            cps.append(pltpu.make_async_copy(
                trg_hbm.at[:, pl.ds(st + i * C, C)], trg_land.at[pp, i],
                pre_sem))
        return cps

    par = lax.rem(b, 2)

    @pl.when(b == 0)
    def _():
        for cp in prefetch_copies(b, par):
            cp.start()

    for cp in prefetch_copies(b, par):
        cp.wait()

    @pl.when(b + 1 < nblk)
    def _():
        for cp in prefetch_copies(b + 1, 1 - par):
            cp.start()

    for i in range(BIGC):
        z_t0 = jnp.transpose(z_land[par, i])
        z_buf[i] = z_t0
        z16_buf[i] = z_t0.astype(jnp.bfloat16)
        ot16_buf[i] = _sel(iota_n, trg_land[par, i]).astype(jnp.bfloat16)

    x_blk_t = jnp.transpose(xn_ref[...])
    c_t = x_blk_t

    for t in range(ROUTE_ITERS):
        cst_t16 = lax.dot_general(
            w1t, c_t, (((1,), (0,)), ((), ())),
            preferred_element_type=jnp.float32).astype(jnp.bfloat16)

        def chunk_math(slot, acc_t):
            z_t = z_buf[slot]
            z16 = z16_buf[slot]
            ot16 = ot16_buf[slot]
            csg_t = lax.dot_general(cst_t16, ot16, (((1,), (0,)), ((), ())),
                                    preferred_element_type=jnp.float32)
            p4 = _gsum(z_t * csg_t)
            e4 = jnp.exp(p4)
            p4n = (e4 / jnp.sum(e4, axis=0, keepdims=True)).astype(jnp.bfloat16)
            w16 = _gexpand(p4n) * z16
            return acc_t + lax.dot_general(
                w16, ot16, (((1,), (1,)), ((), ())),
                preferred_element_type=jnp.float32)

        def pair_body(j, acc_t):
            i0 = 2 * j
            s0 = jnp.where(fits, i0, 0)
            s1 = jnp.where(fits, i0 + 1, 1)

            def stream_dma():
                cps = []
                for li, ii in ((0, i0), (1, i0 + 1)):
                    cps.append(pltpu.make_async_copy(
                        z_hbm.at[pl.ds(start + ii * C, C), :],
                        s_land.at[li], dma_sem))
                    cps.append(pltpu.make_async_copy(
                        trg_hbm.at[:, pl.ds(start + ii * C, C)],
                        strg_land.at[li], dma_sem))
                for cp in cps:
                    cp.start()
                for cp in cps:
                    cp.wait()
                for li, sl in ((0, s0), (1, s1)):
                    z_t = jnp.transpose(s_land[li])
                    z_buf[sl] = z_t
                    z16_buf[sl] = z_t.astype(jnp.bfloat16)
                    ot16_buf[sl] = _sel(iota_n, strg_land[li]).astype(
                        jnp.bfloat16)

            pl.when(jnp.logical_not(fits))(stream_dma)
            acc_t = chunk_math(s0, acc_t)
            return chunk_math(s1, acc_t)

        acc0 = jnp.zeros((D, NB), jnp.float32)
        acc_t = lax.fori_loop(0, npair, pair_body, acc0)
        c_t = x_blk_t + acc_t
        if t < ROUTE_ITERS - 1:
            nrm2 = _gsum(c_t * c_t)
            inv = 1.0 / jnp.maximum(jnp.sqrt(nrm2), 1e-12)
            c_t = c_t * _gexpand(inv)

    out_ref[...] = jnp.transpose(c_t)


def _routing(starts, xn_pad, z, trg_pad2d, npad):
    grid_spec = pltpu.PrefetchScalarGridSpec(
        num_scalar_prefetch=1,
        grid=(npad // NB,),
        in_specs=[
            pl.BlockSpec((NB, D), lambda b, s: (b, 0)),
            pl.BlockSpec(memory_space=pl.ANY),
            pl.BlockSpec(memory_space=pl.ANY),
        ],
        out_specs=pl.BlockSpec((NB, D), lambda b, s: (b, 0)),
        scratch_shapes=[
            pltpu.VMEM((2, BIGC, C, D), jnp.float32),
            pltpu.VMEM((2, BIGC, 1, C), jnp.int32),
            pltpu.VMEM((2, C, D), jnp.float32),
            pltpu.VMEM((2, 1, C), jnp.int32),
            pltpu.VMEM((BIGC, D, C), jnp.float32),
            pltpu.VMEM((BIGC, D, C), jnp.bfloat16),
            pltpu.VMEM((BIGC, NB, C), jnp.bfloat16),
            pltpu.SemaphoreType.DMA,
            pltpu.SemaphoreType.DMA,
        ],
    )
    return pl.pallas_call(
        _routing_body,
        grid_spec=grid_spec,
        out_shape=jax.ShapeDtypeStruct((npad, D), jnp.float32),
        compiler_params=pltpu.CompilerParams(
            dimension_semantics=("arbitrary",)),
    )(starts, xn_pad, z, trg_pad2d)


def kernel(x, edge_index):
    n, d = x.shape
    assert d == D
    src = edge_index[0]
    trg = edge_index[1]
    m = src.shape[0]

    nblk = -(-n // NB)
    npad = nblk * NB
    sc_quant = SC_WORKERS * SC_G
    mpad = -(-(m + BIG) // sc_quant) * sc_quant

    trg_s, src_s = lax.sort((trg, src), num_keys=1, is_stable=False)
    bounds = (jnp.arange(nblk + 1, dtype=jnp.int32) * NB)
    starts = jnp.searchsorted(trg_s, bounds, side="left").astype(jnp.int32)

    pad_m = mpad - m
    src_pad = jnp.concatenate(
        [src_s, (jnp.arange(pad_m, dtype=jnp.int32) % n)])
    trg_pad = jnp.concatenate(
        [trg_s, jnp.full((pad_m,), npad, jnp.int32)]).reshape(1, mpad)

    x_pad = jnp.pad(x, ((0, npad - n), (0, 0)))
    xn_pad = _normalize(x_pad)
    z = _gather_rows(xn_pad, src_pad)
    c = _routing(starts, xn_pad, z, trg_pad, npad)
    return c[:n]
